# baseline (speedup 1.0000x reference)
import functools

import jax
import jax.numpy as jnp
from jax import lax
from jax.experimental import pallas as pl
from jax.experimental.pallas import tpu as pltpu
from jax.experimental.pallas import tpu_sc as plsc

N_FACTORS = 32
BATCH = 16384
NC, NS, L = 2, 16, 16
NW = NC * NS
BPW = BATCH // NW
GROUPS = BPW // L
Y_SCALE = 5.0


def _mf_body(users_hbm, items_hbm, uemb_hbm, vemb_hbm, ubias_hbm, ibias_hbm,
             out_hbm, uidx_v, iidx_v, urows_v, vrows_v, ub_v, ib_v, out_v, sem):
    wid = lax.axis_index("s") * NC + lax.axis_index("c")
    base = wid * BPW

    pltpu.sync_copy(users_hbm.at[pl.ds(base, BPW)], uidx_v)
    pltpu.sync_copy(items_hbm.at[pl.ds(base, BPW)], iidx_v)

    c1 = pltpu.async_copy(uemb_hbm.at[uidx_v], urows_v, sem)
    c2 = pltpu.async_copy(vemb_hbm.at[iidx_v], vrows_v, sem)
    c3 = pltpu.async_copy(ubias_hbm.at[uidx_v], ub_v, sem)
    c4 = pltpu.async_copy(ibias_hbm.at[iidx_v], ib_v, sem)
    c1.wait()
    c2.wait()
    c3.wait()
    c4.wait()

    lane = lax.iota(jnp.int32, L)

    def group(g, _):
        rows = lane + g * L
        acc = ub_v[pl.ds(g * L, L)] + ib_v[pl.ds(g * L, L)]
        for f in range(N_FACTORS):
            cols = jnp.full((L,), f, jnp.int32)
            u = plsc.load_gather(urows_v, [rows, cols])
            v = plsc.load_gather(vrows_v, [rows, cols])
            acc = acc + u * v
        out_v[pl.ds(g * L, L)] = Y_SCALE / (1.0 + jnp.exp(-acc))
        return 0

    lax.fori_loop(0, GROUPS, group, 0, unroll=False)
    pltpu.sync_copy(out_v, out_hbm.at[pl.ds(base, BPW)])


@jax.jit
def _mf_call(users, items, uemb, vemb, ubias, ibias):
    kern = pl.kernel(
        _mf_body,
        out_type=jax.ShapeDtypeStruct((BATCH,), jnp.float32),
        mesh=plsc.VectorSubcoreMesh(core_axis_name="c", subcore_axis_name="s"),
        scratch_types=[
            pltpu.VMEM((BPW,), jnp.int32),
            pltpu.VMEM((BPW,), jnp.int32),
            pltpu.VMEM((BPW, N_FACTORS), jnp.float32),
            pltpu.VMEM((BPW, N_FACTORS), jnp.float32),
            pltpu.VMEM((BPW,), jnp.float32),
            pltpu.VMEM((BPW,), jnp.float32),
            pltpu.VMEM((BPW,), jnp.float32),
            pltpu.SemaphoreType.DMA,
        ],
        compiler_params=pltpu.CompilerParams(use_tc_tiling_on_sc=False, needs_layout_passes=False),
    )
    return kern(users, items, uemb, vemb, ubias, ibias)


def kernel(users, items, user_embedding, item_embedding, user_bias, item_bias):
    users = users.astype(jnp.int32)
    items = items.astype(jnp.int32)
    return _mf_call(users, items, user_embedding, item_embedding,
                    user_bias.reshape(-1), item_bias.reshape(-1))
